# Initial kernel scaffold; baseline (speedup 1.0000x reference)
import jax
import jax.numpy as jnp
from jax.experimental import pallas as pl


def kernel(y, emb, W, b):
    raise NotImplementedError("write your pallas kernel here")


# speedup vs baseline: 1.2065x; 1.2065x over previous
import jax
import jax.numpy as jnp
from jax import lax
from jax.experimental import pallas as pl
from jax.experimental.pallas import tpu as pltpu
from jax.experimental.pallas import tpu_sc as plsc

TDIM = 128
BATCH = 16384

NC = 2
NS = 16
NW = NC * NS
B_PER_W = BATCH // NW
CHUNK = 128
NCHUNK = B_PER_W // CHUNK


def _sc_gather_body(emb_hbm, idx_hbm, out_hbm, idx_v, rows_v, sem):
    wid = lax.axis_index("s") * NC + lax.axis_index("c")
    pltpu.sync_copy(idx_hbm.at[wid], idx_v)
    gathers = [
        pltpu.async_copy(emb_hbm.at[idx_v.at[j]], rows_v.at[j], sem)
        for j in range(NCHUNK)
    ]
    for g in gathers:
        g.wait()
    pltpu.sync_copy(rows_v, out_hbm.at[wid])


def _make_sc_gather():
    mesh = plsc.VectorSubcoreMesh(core_axis_name="c", subcore_axis_name="s")
    return pl.kernel(
        _sc_gather_body,
        out_type=jax.ShapeDtypeStruct((NW, NCHUNK, CHUNK, TDIM), jnp.float32),
        mesh=mesh,
        scratch_types=[
            pltpu.VMEM((NCHUNK, CHUNK), jnp.int32),
            pltpu.VMEM((NCHUNK, CHUNK, TDIM), jnp.float32),
            pltpu.SemaphoreType.DMA,
        ],
    )


@jax.jit
def _sc_gather(emb, y_blocked):
    return _make_sc_gather()(emb, y_blocked)


def _tc_body(h_ref, w_ref, b_ref, o_ref):
    x = h_ref[...]
    s = x * jax.nn.sigmoid(x)
    o_ref[...] = (
        jnp.dot(s, w_ref[...], preferred_element_type=jnp.float32) + b_ref[...]
    )


@jax.jit
def _tc_swish_linear(h, W, b2d):
    bm = 8192
    return pl.pallas_call(
        _tc_body,
        grid=(BATCH // bm,),
        in_specs=[
            pl.BlockSpec((bm, TDIM), lambda i: (i, 0)),
            pl.BlockSpec((TDIM, TDIM), lambda i: (0, 0)),
            pl.BlockSpec((1, TDIM), lambda i: (0, 0)),
        ],
        out_specs=pl.BlockSpec((bm, TDIM), lambda i: (i, 0)),
        out_shape=jax.ShapeDtypeStruct((BATCH, TDIM), jnp.float32),
    )(h, W, b2d)


def kernel(y, emb, W, b):
    y_blocked = y.reshape(NW, NCHUNK, CHUNK)
    h = _sc_gather(emb, y_blocked).reshape(BATCH, TDIM)
    return _tc_swish_linear(h, W, b.reshape(1, TDIM))
